# baseline (speedup 1.0000x reference)
import functools

import jax
import jax.numpy as jnp
from jax import lax
from jax.experimental import pallas as pl
from jax.experimental.pallas import tpu as pltpu
from jax.experimental.pallas import tpu_sc as plsc

_B = 4096
_L = 200
_N = _B * _L
_V = 100001
_T1 = 12
_T2 = 5
_D = 16
_LANES = 16

_NC = 2
_NS = 16

_W = 256
_VP = 100096
_NCHUNK = _VP // _W

_TPT = _N // _NS
_M = 1024
_GSTEPS = _TPT // _M


def _transpose_phase(tab_hbm, dst_hbm, in_stage, out_stage, t_cnt, sid):
    nrows = t_cnt * _LANES
    iota = lax.broadcasted_iota(jnp.int32, (_LANES,), 0)

    def do_chunk(c0, width, ngroups):
        pltpu.sync_copy(tab_hbm.at[pl.ds(0, nrows), pl.ds(c0, width)],
                        in_stage.at[pl.ds(0, nrows), pl.ds(0, width)])

        def per_t(t, c1):
            def per_g(g, c2):
                colbase = g * _LANES
                rowvec = (colbase + iota) * t_cnt + t
                for i in range(_LANES):
                    v = in_stage[t * _LANES + i, pl.ds(colbase, _LANES)]
                    plsc.store_scatter(
                        out_stage, [rowvec, jnp.full((_LANES,), i, jnp.int32)], v)
                return c2

            lax.fori_loop(0, ngroups, per_g, 0)
            return c1

        lax.fori_loop(0, t_cnt, per_t, 0)
        pltpu.sync_copy(out_stage.at[pl.ds(0, width * t_cnt)],
                        dst_hbm.at[pl.ds(c0 * t_cnt, width * t_cnt)])

    def per_chunk(kk, c):
        k = sid + kk * _NS

        @pl.when(k < _NCHUNK)
        def _():
            do_chunk(k * _W, _W, _W // _LANES)

        return c

    lax.fori_loop(0, (_NCHUNK + _NS - 1) // _NS, per_chunk, 0)


def _gather_phase(src_hbm, ids_hbm, sel_hbm, out_hbm, ids_v, sel_v, idx_v,
                  rows_v, sem, t_cnt, col0, sid):
    def step(m, c):
        base = sid * _TPT + m * _M
        pltpu.sync_copy(ids_hbm.at[pl.ds(base, _M)], ids_v)
        pltpu.sync_copy(sel_hbm.at[pl.ds(base, _M)], sel_v)

        def compute(i, c2):
            s = pl.ds(i * _LANES, _LANES)
            idx_v[s] = ids_v[s] * t_cnt + sel_v[s]
            return c2

        lax.fori_loop(0, _M // _LANES, compute, 0)
        pltpu.async_copy(src_hbm.at[idx_v], rows_v, sem).wait()
        pltpu.sync_copy(rows_v, out_hbm.at[pl.ds(base, _M), pl.ds(col0, _D)])
        return c

    lax.fori_loop(0, _GSTEPS, step, 0)


@functools.partial(
    pl.kernel,
    out_type=jax.ShapeDtypeStruct((_N, 2 * _D), jnp.float32),
    mesh=plsc.VectorSubcoreMesh(
        core_axis_name="c", subcore_axis_name="s",
        num_cores=_NC, num_subcores=_NS),
    compiler_params=pltpu.CompilerParams(
        use_tc_tiling_on_sc=False, needs_layout_passes=False),
    scratch_types=[
        pltpu.HBM((_VP * _T1, _D), jnp.float32),
        pltpu.HBM((_VP * _T2, _D), jnp.float32),
        pltpu.VMEM((_T1 * _LANES, _W), jnp.float32),
        pltpu.VMEM((_W * _T1, _D), jnp.float32),
        pltpu.VMEM((_M,), jnp.int32),
        pltpu.VMEM((_M,), jnp.int32),
        pltpu.VMEM((_M,), jnp.int32),
        pltpu.VMEM((_M, _D), jnp.float32),
        pltpu.SemaphoreType.DMA,
    ],
)
def _popularity_gather(log_hbm, t1_hbm, t2_hbm, mtab_hbm, wtab_hbm, out_hbm,
                       mt_hbm, wt_hbm, in_stage, out_stage,
                       ids_v, sel_v, idx_v, rows_v, sem):
    cid = lax.axis_index("c")
    sid = lax.axis_index("s")

    @pl.when(cid == 0)
    def _():
        _transpose_phase(mtab_hbm, mt_hbm, in_stage, out_stage, _T1, sid)

    @pl.when(cid == 1)
    def _():
        _transpose_phase(wtab_hbm, wt_hbm, in_stage, out_stage, _T2, sid)

    plsc.subcore_barrier()

    @pl.when(cid == 0)
    def _():
        _gather_phase(mt_hbm, log_hbm, t1_hbm, out_hbm, ids_v, sel_v, idx_v,
                      rows_v, sem, _T1, 0, sid)

    @pl.when(cid == 1)
    def _():
        _gather_phase(wt_hbm, log_hbm, t2_hbm, out_hbm, ids_v, sel_v, idx_v,
                      rows_v, sem, _T2, _D, sid)


def kernel(log_seqs, time1_seqs, time2_seqs, month_pop_table, week_pop_table):
    log = log_seqs.reshape(_N).astype(jnp.int32)
    t1 = time1_seqs.reshape(_N).astype(jnp.int32)
    t2 = time2_seqs.reshape(_N).astype(jnp.int32)
    mtab = jnp.pad(month_pop_table, ((0, 0), (0, _VP - _V)))
    wtab = jnp.pad(week_pop_table, ((0, 0), (0, _VP - _V)))
    out = _popularity_gather(log, t1, t2, mtab, wtab)
    return out.reshape(_B, _L, 2 * _D)


# speedup vs baseline: 1.2137x; 1.2137x over previous
import functools

import jax
import jax.numpy as jnp
from jax import lax
from jax.experimental import pallas as pl
from jax.experimental.pallas import tpu as pltpu
from jax.experimental.pallas import tpu_sc as plsc

_B = 4096
_L = 200
_N = _B * _L
_V = 100001
_T1 = 12
_T2 = 5
_D = 16
_LANES = 16

_NC = 2
_NS = 16

_W = 128
_VP = 100096
_NCHUNK = _VP // _W
_NSLOT = 2 * ((_NCHUNK + 2 * _NS - 1) // (2 * _NS))

_TPT = _N // _NS
_M = 1024
_GSTEPS = _TPT // _M


def _transpose_phase(tab_hbm, dst_hbm, in_bufs, out_bufs, sem_in, sem_out,
                     t_cnt, sid):
    nrows = t_cnt * _LANES
    iota = lax.broadcasted_iota(jnp.int32, (_LANES,), 0)

    def in_slice(k):
        return tab_hbm.at[pl.ds(0, nrows), pl.ds(k * _W, _W)]

    def in_buf(b):
        return in_bufs[b].at[pl.ds(0, nrows), :]

    def out_slice(k):
        return dst_hbm.at[pl.ds(k * _W * t_cnt, _W * t_cnt)]

    def out_buf(b):
        return out_bufs[b].at[pl.ds(0, _W * t_cnt)]

    def transpose_chunk(b):
        in_stage, out_stage = in_bufs[b], out_bufs[b]

        def per_t(t, c1):
            def per_g(g, c2):
                colbase = g * _LANES
                rowvec = (colbase + iota) * t_cnt + t
                vals = [in_stage[t * _LANES + i, pl.ds(colbase, _LANES)]
                        for i in range(_LANES)]
                for i in range(_LANES):
                    plsc.store_scatter(
                        out_stage,
                        [rowvec, jnp.full((_LANES,), i, jnp.int32)], vals[i])
                return c2

            lax.fori_loop(0, _W // _LANES, per_g, 0)
            return c1

        lax.fori_loop(0, t_cnt, per_t, 0)

    pltpu.async_copy(in_slice(sid), in_buf(0), sem_in)

    def ring(kk2, c):
        for b in (0, 1):
            kk = kk2 * 2 + b
            k = sid + kk * _NS

            @pl.when(k < _NCHUNK)
            def _():
                pltpu.make_async_copy(in_slice(k), in_buf(b), sem_in).wait()

                @pl.when(sid + (kk + 1) * _NS < _NCHUNK)
                def _():
                    pltpu.async_copy(
                        in_slice(sid + (kk + 1) * _NS), in_buf(1 - b), sem_in)

                @pl.when(kk >= 2)
                def _():
                    pltpu.make_async_copy(
                        out_buf(b), out_slice(k), sem_out).wait()

                transpose_chunk(b)
                pltpu.async_copy(out_buf(b), out_slice(k), sem_out)

        return c

    lax.fori_loop(0, _NSLOT // 2, ring, 0)
    pltpu.make_async_copy(out_buf(0), out_slice(0), sem_out).wait()
    pltpu.make_async_copy(out_buf(1), out_slice(0), sem_out).wait()


def _gather_phase(src_hbm, ids_hbm, sel_hbm, out_hbm, ids_v, sel_v, idx_v,
                  rows_v, sem, t_cnt, col0, sid):
    def step(m, c):
        base = sid * _TPT + m * _M
        pltpu.sync_copy(ids_hbm.at[pl.ds(base, _M)], ids_v)
        pltpu.sync_copy(sel_hbm.at[pl.ds(base, _M)], sel_v)

        def compute(i, c2):
            s = pl.ds(i * _LANES, _LANES)
            idx_v[s] = ids_v[s] * t_cnt + sel_v[s]
            return c2

        lax.fori_loop(0, _M // _LANES, compute, 0)
        pltpu.async_copy(src_hbm.at[idx_v], rows_v, sem).wait()
        pltpu.sync_copy(rows_v, out_hbm.at[pl.ds(base, _M), pl.ds(col0, _D)])
        return c

    lax.fori_loop(0, _GSTEPS, step, 0)


@functools.partial(
    pl.kernel,
    out_type=jax.ShapeDtypeStruct((_N, 2 * _D), jnp.float32),
    mesh=plsc.VectorSubcoreMesh(
        core_axis_name="c", subcore_axis_name="s",
        num_cores=_NC, num_subcores=_NS),
    compiler_params=pltpu.CompilerParams(
        use_tc_tiling_on_sc=False, needs_layout_passes=False),
    scratch_types=[
        pltpu.HBM((_VP * _T1, _D), jnp.float32),
        pltpu.HBM((_VP * _T2, _D), jnp.float32),
        pltpu.VMEM((_T1 * _LANES, _W), jnp.float32),
        pltpu.VMEM((_T1 * _LANES, _W), jnp.float32),
        pltpu.VMEM((_W * _T1, _D), jnp.float32),
        pltpu.VMEM((_W * _T1, _D), jnp.float32),
        pltpu.VMEM((_M,), jnp.int32),
        pltpu.VMEM((_M,), jnp.int32),
        pltpu.VMEM((_M,), jnp.int32),
        pltpu.VMEM((_M, _D), jnp.float32),
        pltpu.SemaphoreType.DMA,
        pltpu.SemaphoreType.DMA,
        pltpu.SemaphoreType.DMA,
    ],
)
def _popularity_gather(log_hbm, t1_hbm, t2_hbm, mtab_hbm, wtab_hbm, out_hbm,
                       mt_hbm, wt_hbm, in0, in1, ost0, ost1,
                       ids_v, sel_v, idx_v, rows_v, sem, sem_in, sem_out):
    cid = lax.axis_index("c")
    sid = lax.axis_index("s")

    @pl.when(cid == 0)
    def _():
        _transpose_phase(mtab_hbm, mt_hbm, (in0, in1), (ost0, ost1),
                         sem_in, sem_out, _T1, sid)

    @pl.when(cid == 1)
    def _():
        _transpose_phase(wtab_hbm, wt_hbm, (in0, in1), (ost0, ost1),
                         sem_in, sem_out, _T2, sid)

    plsc.subcore_barrier()

    @pl.when(cid == 0)
    def _():
        _gather_phase(mt_hbm, log_hbm, t1_hbm, out_hbm, ids_v, sel_v, idx_v,
                      rows_v, sem, _T1, 0, sid)

    @pl.when(cid == 1)
    def _():
        _gather_phase(wt_hbm, log_hbm, t2_hbm, out_hbm, ids_v, sel_v, idx_v,
                      rows_v, sem, _T2, _D, sid)


def kernel(log_seqs, time1_seqs, time2_seqs, month_pop_table, week_pop_table):
    log = log_seqs.reshape(_N).astype(jnp.int32)
    t1 = time1_seqs.reshape(_N).astype(jnp.int32)
    t2 = time2_seqs.reshape(_N).astype(jnp.int32)
    mtab = jnp.pad(month_pop_table, ((0, 0), (0, _VP - _V)))
    wtab = jnp.pad(week_pop_table, ((0, 0), (0, _VP - _V)))
    out = _popularity_gather(log, t1, t2, mtab, wtab)
    return out.reshape(_B, _L, 2 * _D)
